# baseline (speedup 1.0000x reference)
import functools

import jax
import jax.numpy as jnp
from jax import lax
from jax.experimental import pallas as pl
from jax.experimental.pallas import tpu as pltpu
from jax.experimental.pallas import tpu_sc as plsc

_L = 16


def kernel(h, edge_index):
    n_nodes, d_feat = h.shape
    n_edges = edge_index.shape[1]
    nw2 = d_feat // 2
    nkb = d_feat // (2 * _L)
    info = plsc.get_sparse_core_info()
    nc, ns = info.num_cores, info.num_subcores
    nw = nc * ns
    assert n_edges % nw == 0
    epw = n_edges // nw
    C = 80
    assert epw % C == 0 and C % _L == 0
    nchunk = epw // C
    ngroup = C // _L
    npad = nchunk + (nchunk % 2)
    rpt = n_nodes // ns
    RP = 125
    assert rpt % RP == 0

    mesh = plsc.VectorSubcoreMesh(core_axis_name="c", subcore_axis_name="s")

    @functools.partial(
        pl.kernel,
        mesh=mesh,
        compiler_params=pltpu.CompilerParams(
            needs_layout_passes=False, use_tc_tiling_on_sc=False),
        out_type=jax.ShapeDtypeStruct((nw, epw), jnp.float32),
        scratch_types=[
            pltpu.VMEM((epw + C,), jnp.int32),
            pltpu.VMEM((epw + C,), jnp.int32),
            pltpu.VMEM((C, nw2), jnp.int32),
            pltpu.VMEM((C, nw2), jnp.int32),
            pltpu.VMEM((C, nw2), jnp.int32),
            pltpu.VMEM((C, nw2), jnp.int32),
            pltpu.VMEM((epw + C,), jnp.float32),
            pltpu.VMEM((C * _L,), jnp.float32),
            pltpu.VMEM((125, d_feat), jnp.float32),
            pltpu.VMEM((125, nw2), jnp.int32),
            pltpu.VMEM_SHARED((10000, 64), jnp.int32),
            pltpu.SemaphoreType.DMA,
            pltpu.SemaphoreType.DMA,
        ],
    )
    def k(h_hbm, e_hbm, out_hbm,
          sidx_all, didx_all, srows0, drows0, srows1, drows1, outv_all,
          pmat, pk_in, pk_out, shared_h, sem0, sem1):
        sid = lax.axis_index("s")
        wid = sid * nc + lax.axis_index("c")
        base0 = wid * epw
        colbase = lax.iota(jnp.int32, _L) * _L
        RP = 125

        for pc in range(rpt // RP):
            r0 = sid * rpt + pc * RP
            pltpu.sync_copy(h_hbm.at[pl.ds(r0, RP)], pk_in)

            @plsc.parallel_loop(0, RP, unroll=5)
            def row_body(r):
                for q in range(nkb):
                    a = pk_in[r, pl.ds(q * 2 * _L, _L)]
                    b = pk_in[r, pl.ds(q * 2 * _L + _L, _L)]
                    w = plsc.pack(a, b, format=plsc.PackFormat.INTERLEAVED)
                    pk_out[r, pl.ds(q * _L, _L)] = plsc.bitcast(w, jnp.int32)
            pltpu.sync_copy(pk_out, shared_h.at[pl.ds(r0, RP)])

        pltpu.sync_copy(e_hbm.at[0, pl.ds(base0, epw)],
                        sidx_all.at[pl.ds(0, epw)])
        pltpu.sync_copy(e_hbm.at[1, pl.ds(base0, epw)],
                        didx_all.at[pl.ds(0, epw)])
        if npad != nchunk:
            for g in range(ngroup):
                sidx_all[pl.ds((nchunk * C) + g * _L, _L)] = (
                    sidx_all[pl.ds(g * _L, _L)])
                didx_all[pl.ds((nchunk * C) + g * _L, _L)] = (
                    didx_all[pl.ds(g * _L, _L)])
        plsc.subcore_barrier()

        bufs = ((srows0, drows0, sem0), (srows1, drows1, sem1))

        def start(c, b):
            srows, drows, sem = bufs[b]
            pltpu.async_copy(
                shared_h.at[sidx_all.at[pl.ds(c * C, C)]], srows, sem)
            pltpu.async_copy(
                shared_h.at[didx_all.at[pl.ds(c * C, C)]], drows, sem)

        def wait(b):
            srows, drows, sem = bufs[b]
            pltpu.make_async_copy(
                shared_h.at[sidx_all.at[pl.ds(0, C)]], srows, sem).wait()
            pltpu.make_async_copy(
                shared_h.at[didx_all.at[pl.ds(0, C)]], drows, sem).wait()

        def compute(c, b):
            srows, drows, _ = bufs[b]

            @plsc.parallel_loop(0, C, unroll=4)
            def edge_body(i):
                acc = None
                for kk in range(nkb):
                    sv = plsc.bitcast(srows[i, pl.ds(kk * _L, _L)],
                                      jnp.bfloat16)
                    dv = plsc.bitcast(drows[i, pl.ds(kk * _L, _L)],
                                      jnp.bfloat16)
                    pr = sv * dv
                    acc = pr if acc is None else acc + pr
                pa, pb = plsc.unpack(acc, format=plsc.PackFormat.INTERLEAVED)
                pmat[pl.ds(i * _L, _L)] = pa + pb

            @plsc.parallel_loop(0, ngroup, unroll=5)
            def red_body(g):
                pb_ = g * _L * _L
                tot = plsc.load_gather(pmat, [pb_ + colbase])
                for l in range(1, _L):
                    tot = tot + plsc.load_gather(pmat, [pb_ + colbase + l])
                outv_all[pl.ds(c * C + g * _L, _L)] = tot

        start(0, 0)
        start(1, 1)

        def pair_body(cc, carry):
            c0 = 2 * cc
            wait(0)
            compute(c0, 0)

            @pl.when(c0 + 2 < npad)
            def _p0():
                start(c0 + 2, 0)

            wait(1)
            compute(c0 + 1, 1)

            @pl.when(c0 + 3 < npad)
            def _p1():
                start(c0 + 3, 1)

            return carry

        lax.fori_loop(0, npad // 2, pair_body, 0)

        pltpu.sync_copy(outv_all.at[pl.ds(0, epw)], out_hbm.at[wid])

    return k(h, edge_index).reshape(-1, 1)


# speedup vs baseline: 1.0766x; 1.0766x over previous
import functools

import jax
import jax.numpy as jnp
from jax import lax
from jax.experimental import pallas as pl
from jax.experimental.pallas import tpu as pltpu
from jax.experimental.pallas import tpu_sc as plsc

_L = 16


def kernel(h, edge_index):
    n_nodes, d_feat = h.shape
    n_edges = edge_index.shape[1]
    nw2 = d_feat // 2
    nkb = d_feat // (2 * _L)
    info = plsc.get_sparse_core_info()
    nc, ns = info.num_cores, info.num_subcores
    nw = nc * ns
    assert n_edges % nw == 0
    epw = n_edges // nw
    C = 80
    assert epw % C == 0 and C % _L == 0
    nchunk = epw // C
    ngroup = C // _L
    npad = nchunk + (nchunk % 2)
    rpt = n_nodes // ns
    RP = 125
    assert rpt % RP == 0

    mesh = plsc.VectorSubcoreMesh(core_axis_name="c", subcore_axis_name="s")

    @functools.partial(
        pl.kernel,
        mesh=mesh,
        compiler_params=pltpu.CompilerParams(
            needs_layout_passes=False, use_tc_tiling_on_sc=False),
        out_type=jax.ShapeDtypeStruct((nw, epw), jnp.float32),
        scratch_types=[
            pltpu.VMEM((epw + C,), jnp.int32),
            pltpu.VMEM((epw + C,), jnp.int32),
            pltpu.VMEM((C, nw2), jnp.int32),
            pltpu.VMEM((C, nw2), jnp.int32),
            pltpu.VMEM((C, nw2), jnp.int32),
            pltpu.VMEM((C, nw2), jnp.int32),
            pltpu.VMEM((epw + C,), jnp.float32),
            pltpu.VMEM((C * _L,), jnp.float32),
            pltpu.VMEM((125, d_feat), jnp.float32),
            pltpu.VMEM((125, nw2), jnp.int32),
            pltpu.VMEM_SHARED((10000, 64), jnp.int32),
            pltpu.SemaphoreType.DMA,
            pltpu.SemaphoreType.DMA,
        ],
    )
    def k(h_hbm, e_hbm, out_hbm,
          sidx_all, didx_all, srows0, drows0, srows1, drows1, outv_all,
          pmat, pk_in, pk_out, shared_h, sem0, sem1):
        sid = lax.axis_index("s")
        wid = sid * nc + lax.axis_index("c")
        base0 = wid * epw
        colbase = lax.iota(jnp.int32, _L) * _L
        RP = 125

        for pc in range(rpt // RP):
            r0 = sid * rpt + pc * RP
            pltpu.sync_copy(h_hbm.at[pl.ds(r0, RP)], pk_in)

            @plsc.parallel_loop(0, RP, unroll=5)
            def row_body(r):
                for q in range(nkb):
                    a = pk_in[r, pl.ds(q * 2 * _L, _L)]
                    b = pk_in[r, pl.ds(q * 2 * _L + _L, _L)]
                    w = plsc.pack(a, b, format=plsc.PackFormat.INTERLEAVED)
                    pk_out[r, pl.ds(q * _L, _L)] = plsc.bitcast(w, jnp.int32)
            pltpu.sync_copy(pk_out, shared_h.at[pl.ds(r0, RP)])

        pltpu.sync_copy(e_hbm.at[0, pl.ds(base0, epw)],
                        sidx_all.at[pl.ds(0, epw)])
        pltpu.sync_copy(e_hbm.at[1, pl.ds(base0, epw)],
                        didx_all.at[pl.ds(0, epw)])
        if npad != nchunk:
            for g in range(ngroup):
                sidx_all[pl.ds((nchunk * C) + g * _L, _L)] = (
                    sidx_all[pl.ds(g * _L, _L)])
                didx_all[pl.ds((nchunk * C) + g * _L, _L)] = (
                    didx_all[pl.ds(g * _L, _L)])
        plsc.subcore_barrier()

        bufs = ((srows0, drows0, sem0), (srows1, drows1, sem1))

        def start(c, b):
            srows, drows, sem = bufs[b]
            pltpu.async_copy(
                shared_h.at[sidx_all.at[pl.ds(c * C, C)]], srows, sem)
            pltpu.async_copy(
                shared_h.at[didx_all.at[pl.ds(c * C, C)]], drows, sem)

        def wait(b):
            srows, drows, sem = bufs[b]
            pltpu.make_async_copy(
                shared_h.at[sidx_all.at[pl.ds(0, C)]], srows, sem).wait()
            pltpu.make_async_copy(
                shared_h.at[didx_all.at[pl.ds(0, C)]], drows, sem).wait()

        def compute(c, b):
            srows, drows, _ = bufs[b]

            @plsc.parallel_loop(0, C, unroll=4)
            def edge_body(i):
                acc = None
                for kk in range(nkb):
                    sv = plsc.bitcast(srows[i, pl.ds(kk * _L, _L)],
                                      jnp.bfloat16)
                    dv = plsc.bitcast(drows[i, pl.ds(kk * _L, _L)],
                                      jnp.bfloat16)
                    pr = sv * dv
                    acc = pr if acc is None else acc + pr
                pa, pb = plsc.unpack(acc, format=plsc.PackFormat.INTERLEAVED)
                pmat[pl.ds(i * _L, _L)] = pa + pb

            @plsc.parallel_loop(0, ngroup, unroll=1)
            def red_body(g):
                pb_ = g * _L * _L
                tot = plsc.load_gather(pmat, [pb_ + colbase])
                for l in range(1, _L):
                    tot = tot + plsc.load_gather(pmat, [pb_ + colbase + l])
                outv_all[pl.ds(c * C + g * _L, _L)] = tot

        start(0, 0)
        start(1, 1)

        def pair_body(cc, carry):
            c0 = 2 * cc
            wait(0)
            compute(c0, 0)

            @pl.when(c0 + 2 < npad)
            def _p0():
                start(c0 + 2, 0)

            wait(1)
            compute(c0 + 1, 1)

            @pl.when(c0 + 3 < npad)
            def _p1():
                start(c0 + 3, 1)

            return carry

        lax.fori_loop(0, npad // 2, pair_body, 0)

        pltpu.sync_copy(outv_all.at[pl.ds(0, epw)], out_hbm.at[wid])

    return k(h, edge_index).reshape(-1, 1)
